# Initial kernel scaffold; baseline (speedup 1.0000x reference)
import jax
import jax.numpy as jnp
from jax.experimental import pallas as pl


def kernel(idx, targets, tok_table, pos_table, W, b):
    raise NotImplementedError("write your pallas kernel here")


# speedup vs baseline: 1.7589x; 1.7589x over previous
import jax
import jax.numpy as jnp
from jax.experimental import pallas as pl

VOCAB = 1000
N_EMBD = 32
B = 4096
T = 8
BT = B * T
R = 256
NB = BT // R


def _body(idx_ref, tgt_ref, tok_ref, w_ref, b_ref, out_ref, loss_ref):
    i = pl.program_id(0)
    idxb = idx_ref[0]
    tgtb = tgt_ref[0]
    iota = jax.lax.broadcasted_iota(jnp.int32, (R, VOCAB), 1)
    oh = (idxb == iota).astype(jnp.float32)
    emb = jnp.dot(oh, tok_ref[...], preferred_element_type=jnp.float32)
    logits = (
        jnp.dot(emb, w_ref[...], preferred_element_type=jnp.float32) + b_ref[...]
    )
    out_ref[...] = logits

    m = jnp.max(logits, axis=1, keepdims=True)
    lse = jnp.log(jnp.sum(jnp.exp(logits - m), axis=1, keepdims=True)) + m
    picked = jnp.sum(
        jnp.where(tgtb == iota, logits, 0.0), axis=1, keepdims=True
    )
    partial = jnp.sum(lse - picked).reshape(1, 1)

    @pl.when(i == 0)
    def _():
        loss_ref[...] = jnp.zeros((1, 1), jnp.float32)

    loss_ref[...] += partial


def kernel(idx, targets, tok_table, pos_table, W, b):
    del pos_table
    idx_r = idx.reshape(NB, R, 1).astype(jnp.int32)
    tgt_r = targets.reshape(NB, R, 1).astype(jnp.int32)
    b2 = b.reshape(1, VOCAB)

    logits_flat, loss_acc = pl.pallas_call(
        _body,
        grid=(NB,),
        in_specs=[
            pl.BlockSpec((1, R, 1), lambda i: (i, 0, 0)),
            pl.BlockSpec((1, R, 1), lambda i: (i, 0, 0)),
            pl.BlockSpec((VOCAB, N_EMBD), lambda i: (0, 0)),
            pl.BlockSpec((N_EMBD, VOCAB), lambda i: (0, 0)),
            pl.BlockSpec((1, VOCAB), lambda i: (0, 0)),
        ],
        out_specs=[
            pl.BlockSpec((R, VOCAB), lambda i: (i, 0)),
            pl.BlockSpec((1, 1), lambda i: (0, 0)),
        ],
        out_shape=[
            jax.ShapeDtypeStruct((BT, VOCAB), jnp.float32),
            jax.ShapeDtypeStruct((1, 1), jnp.float32),
        ],
    )(idx_r, tgt_r, tok_table, W, b2)

    loss = (loss_acc[0, 0] / BT).astype(jnp.float32)
    return (logits_flat, loss)
